# baseline (speedup 1.0000x reference)
import functools

import jax
import jax.numpy as jnp
import numpy as np
from jax import lax
from jax.experimental import pallas as pl
from jax.experimental.pallas import tpu as pltpu
from jax.experimental.pallas import tpu_sc as plsc

N = 10000
E = 160000
D = 256
H = 128
NT = 16
NP = 10240
STRIDE = NP // NT
EPT = E // NT
CH = 64
NCH = 160
ECT = NCH * CH
EP = NT * ECT
NRING = 2
NIDX = 4

_mesh = plsc.VectorSubcoreMesh(core_axis_name="c", subcore_axis_name="s")

_sc_params = pltpu.CompilerParams(needs_layout_passes=False)
_sc_params_untiled = pltpu.CompilerParams(needs_layout_passes=False,
                                          use_tc_tiling_on_sc=False)

_COLMAP = np.array(
    [32 * (m // 32) + ((m % 32) // 2 if m % 2 == 0 else 16 + (m % 32 - 1) // 2)
     for m in range(H)], dtype=np.int32)
_PERM256 = np.concatenate([_COLMAP, _COLMAP + H])


def _rsqrt16(x):
    x = jnp.maximum(x, 1.0)
    i = plsc.bitcast(x, jnp.int32)
    i = 0x5F3759DF - lax.shift_right_logical(i, 1)
    y = plsc.bitcast(i, jnp.float32)
    for _ in range(3):
        y = y * (1.5 - 0.5 * x * y * y)
    return y


@functools.partial(
    pl.kernel,
    out_type=[
        jax.ShapeDtypeStruct((NP,), jnp.float32),
        jax.ShapeDtypeStruct((EP,), jnp.float32),
    ],
    mesh=_mesh,
    scratch_types=[
        pltpu.VMEM((NP,), jnp.float32),
        pltpu.VMEM((ECT,), jnp.int32),
        pltpu.VMEM((ECT,), jnp.float32),
        pltpu.VMEM((STRIDE,), jnp.float32),
        pltpu.VMEM((STRIDE,), jnp.float32),
        pltpu.VMEM_SHARED((NT * NP,), jnp.float32),
    ],
    compiler_params=_sc_params,
)
def _prep_kernel(idx_hbm, srcf_hbm, wf_hbm, nd_hbm, c_hbm,
                 hist_v, idx_v, w_v, acc_v, tmp_v, shared):
    cx = lax.axis_index("c")
    s = lax.axis_index("s")
    pltpu.sync_copy(idx_hbm.at[pl.ds((cx * NT + s) * EPT, EPT)],
                    idx_v.at[pl.ds(0, EPT)])

    @pl.loop(0, NP, step=16)
    def _zero(i):
        hist_v[pl.ds(i, 16)] = jnp.zeros((16,), jnp.float32)

    @pl.loop(0, EPT, step=16)
    def _count(e0):
        idx16 = idx_v[pl.ds(e0, 16)]
        cnt, last = plsc.scan_count(idx16)
        plsc.addupdate_scatter(hist_v, [idx16], cnt.astype(jnp.float32),
                               mask=last)

    pltpu.sync_copy(hist_v, shared.at[pl.ds(s * NP, NP)])
    plsc.subcore_barrier()

    base = s * STRIDE

    @pl.loop(0, STRIDE, step=16)
    def _zacc(i):
        acc_v[pl.ds(i, 16)] = jnp.zeros((16,), jnp.float32)

    @pl.loop(0, NT)
    def _sum(t):
        pltpu.sync_copy(shared.at[pl.ds(t * NP + base, STRIDE)], tmp_v)

        @pl.loop(0, STRIDE, step=16)
        def _add(i):
            acc_v[pl.ds(i, 16)] = acc_v[pl.ds(i, 16)] + tmp_v[pl.ds(i, 16)]

    @pl.loop(0, STRIDE, step=16)
    def _norm(i):
        acc_v[pl.ds(i, 16)] = _rsqrt16(acc_v[pl.ds(i, 16)])

    plsc.subcore_barrier()

    @pl.when(cx == 1)
    def _():
        pltpu.sync_copy(acc_v, nd_hbm.at[pl.ds(base, STRIDE)])

    @pl.when(cx == 0)
    def _():
        pltpu.sync_copy(acc_v, shared.at[pl.ds(base, STRIDE)])

    plsc.subcore_barrier()

    @pl.when(cx == 0)
    def _():
        pltpu.sync_copy(shared.at[pl.ds(0, NP)], hist_v)
        pltpu.sync_copy(srcf_hbm.at[pl.ds(s * ECT, ECT)], idx_v)
        pltpu.sync_copy(wf_hbm.at[pl.ds(s * ECT, ECT)], w_v)

        @pl.loop(0, ECT, step=16)
        def _coef(e0):
            s16 = idx_v[pl.ds(e0, 16)]
            ns16 = plsc.load_gather(hist_v, [s16])
            w_v[pl.ds(e0, 16)] = w_v[pl.ds(e0, 16)] * ns16

        pltpu.sync_copy(w_v, c_hbm.at[pl.ds(s * ECT, ECT)])


@functools.partial(
    pl.kernel,
    out_type=jax.ShapeDtypeStruct((2 * NP, H), jnp.float32),
    mesh=_mesh,
    scratch_types=[
        pltpu.VMEM((NCH, CH), jnp.int32),
        pltpu.VMEM((NIDX, CH), jnp.int32),
        pltpu.VMEM((NIDX, CH), jnp.float32),
        [pltpu.VMEM((CH, H // 2), jnp.float32)] * NRING,
        [pltpu.VMEM((CH, H), jnp.float32)] * NRING,
        pltpu.VMEM_SHARED((NP, H), jnp.float32),
        [pltpu.SemaphoreType.DMA] * NRING,
        [pltpu.SemaphoreType.DMA] * NRING,
    ],
    compiler_params=_sc_params_untiled,
)
def _spmm_kernel(x_hbm, src_hbm, dst_hbm, w_hbm, zero_hbm, out_hbm,
                 src_v, dst_v, w_v, rowsb, rowsf, acc, gsem, ssem):
    c = lax.axis_index("c")
    s = lax.axis_index("s")
    pltpu.sync_copy(src_hbm.at[c * NT + s], src_v)

    def gstart(gb, db, q):
        pltpu.async_copy(x_hbm.at[src_v.at[q]], rowsb[gb], gsem[gb])
        off = (s * NCH + q) * CH
        pltpu.async_copy(dst_hbm.at[pl.ds(off, CH)], dst_v.at[db], gsem[gb])
        pltpu.async_copy(w_hbm.at[pl.ds(off, CH)], w_v.at[db], gsem[gb])

    def gwait(gb, db, q):
        pltpu.make_async_copy(x_hbm.at[src_v.at[q]], rowsb[gb],
                              gsem[gb]).wait()
        off = (s * NCH + q) * CH
        pltpu.make_async_copy(dst_hbm.at[pl.ds(off, CH)], dst_v.at[db],
                              gsem[gb]).wait()
        pltpu.make_async_copy(w_hbm.at[pl.ds(off, CH)], w_v.at[db],
                              gsem[gb]).wait()

    def sstart(fb, db):
        pltpu.async_copy(rowsf[fb], acc.at[dst_v.at[db]], ssem[fb], add=True)

    def swait(fb, db):
        pltpu.make_async_copy(rowsf[fb], acc.at[dst_v.at[db]],
                              ssem[fb]).wait()

    gstart(0, 0, 0)
    gstart(1, 1, 1)

    pltpu.sync_copy(zero_hbm.at[pl.ds(s * STRIDE, STRIDE)],
                    acc.at[pl.ds(s * STRIDE, STRIDE)])
    plsc.subcore_barrier()

    @pl.loop(0, NCH, step=NIDX)
    def _iter(j):
        for k in range(NIDX):
            q = j + k
            gb = fb = k % NRING
            db = k
            gwait(gb, db, q)
            if k < 2:
                @pl.when(j > 0)
                def _():
                    swait(fb, (k + 2) % NIDX)
            else:
                swait(fb, (k + 2) % NIDX)

            @pl.loop(0, CH, step=16)
            def _scale(r0):
                wv = w_v[db, pl.ds(r0, 16)]
                for r in range(16):
                    wr = wv[r]
                    for g in range(H // 32):
                        v = rowsb[gb][r0 + r, pl.ds(g * 16, 16)]
                        v32 = plsc.bitcast(v, jnp.bfloat16)
                        lo, hi = plsc.unpack(
                            v32, format=plsc.PackFormat.INTERLEAVED)
                        rowsf[fb][r0 + r, pl.ds(g * 32, 16)] = lo * wr
                        rowsf[fb][r0 + r, pl.ds(g * 32 + 16, 16)] = hi * wr

            sstart(fb, db)
            ndb = (k + 2) % NIDX
            if k < 2:
                gstart(gb, ndb, q + 2)
            else:
                @pl.when(j < NCH - NIDX)
                def _():
                    gstart(gb, ndb, q + 2)

    swait(0, (NCH - 2) % NIDX)
    swait(1, (NCH - 1) % NIDX)
    plsc.subcore_barrier()

    pltpu.sync_copy(acc.at[pl.ds(s * STRIDE, STRIDE)],
                    out_hbm.at[pl.ds(c * NP + s * STRIDE, STRIDE)])


BN = 1280
BNF = 1000


def _mm1_body(x_ref, w_ref, o_ref):
    x = x_ref[...]
    w = w_ref[...]
    o_ref[0] = jnp.dot(x, w[:, :H],
                       preferred_element_type=jnp.float32).astype(jnp.bfloat16)
    o_ref[1] = jnp.dot(x, w[:, H:],
                       preferred_element_type=jnp.float32).astype(jnp.bfloat16)


def _mm2_body(a_ref, nd_ref, b_ref, w_ref, o_ref):
    nd = nd_ref[...]
    b = b_ref[...]
    ha = jnp.maximum(a_ref[0] * nd + b[0, :H], 0.0)
    hb = jnp.maximum(a_ref[1] * nd + b[0, H:], 0.0)
    w = w_ref[...]
    o_ref[0] = (jnp.dot(ha, w[:H, :H], preferred_element_type=jnp.float32)
                + jnp.dot(hb, w[H:, :H], preferred_element_type=jnp.float32)
                ).astype(jnp.bfloat16)
    o_ref[1] = (jnp.dot(ha, w[:H, H:], preferred_element_type=jnp.float32)
                + jnp.dot(hb, w[H:, H:], preferred_element_type=jnp.float32)
                ).astype(jnp.bfloat16)


def _fin_body(a_ref, nd_ref, b_ref, o_ref):
    nd = nd_ref[...]
    b = b_ref[...]
    o_ref[:, :H] = a_ref[0] * nd + b[0, :H]
    o_ref[:, H:] = a_ref[1] * nd + b[0, H:]


_row_spec = pl.BlockSpec((BN, D), lambda i: (i, 0))
_halves_spec = pl.BlockSpec((2, BN, H), lambda i: (0, i, 0))
_nd_spec = pl.BlockSpec((BN, 1), lambda i: (i, 0))
_w_spec = pl.BlockSpec((D, D), lambda i: (0, 0))
_b_spec = pl.BlockSpec((1, D), lambda i: (0, 0))

_mm1 = pl.pallas_call(
    _mm1_body,
    grid=(NP // BN,),
    in_specs=[_row_spec, _w_spec],
    out_specs=_halves_spec,
    out_shape=jax.ShapeDtypeStruct((2, NP, H), jnp.bfloat16),
)

_mm2 = pl.pallas_call(
    _mm2_body,
    grid=(NP // BN,),
    in_specs=[_halves_spec, _nd_spec, _b_spec, _w_spec],
    out_specs=_halves_spec,
    out_shape=jax.ShapeDtypeStruct((2, NP, H), jnp.bfloat16),
)

_fin = pl.pallas_call(
    _fin_body,
    grid=(N // BNF,),
    in_specs=[pl.BlockSpec((2, BNF, H), lambda i: (0, i, 0)),
              pl.BlockSpec((BNF, 1), lambda i: (i, 0)),
              _b_spec],
    out_specs=pl.BlockSpec((BNF, D), lambda i: (i, 0)),
    out_shape=jax.ShapeDtypeStruct((N, D), jnp.float32),
)


def kernel(features, edge_index, edge_weight, W1, b1, W2, b2):
    src = edge_index[0]
    dst = edge_index[1]

    idx_flat = jnp.concatenate([src, dst])
    pad = EP - E
    pad_idx = (jnp.arange(pad, dtype=jnp.int32) * 37) % N
    srcf = jnp.concatenate([src, pad_idx])
    srcp = srcf.reshape(NT, NCH, CH)
    srcp2 = jnp.concatenate([srcp, srcp + NP], axis=0)
    dstp = jnp.concatenate([dst, pad_idx])
    wp = jnp.concatenate([edge_weight, jnp.zeros((pad,), jnp.float32)])
    xpad = jnp.pad(features, ((0, NP - N), (0, 0)))
    zeros_half = jnp.zeros((NP, H), jnp.float32)
    b1r = b1.reshape(1, D)
    b2r = b2.reshape(1, D)

    W1p = W1[:, _PERM256]
    W2p = W2[:, _PERM256]

    nd, cw = _prep_kernel(idx_flat, srcf, wp)
    ndr = nd.reshape(NP, 1)
    def _as_packed(x):
        return lax.bitcast_convert_type(
            x.reshape(2 * NP, H // 2, 2), jnp.float32)

    x1 = _mm1(xpad, W1p)
    a1 = _spmm_kernel(_as_packed(x1), srcp2, dstp, cw, zeros_half)
    x2 = _mm2(a1.reshape(2, NP, H), ndr, b1r, W2p)
    a2 = _spmm_kernel(_as_packed(x2), srcp2, dstp, cw, zeros_half)
    return _fin(a2.reshape(2, NP, H), ndr[:N], b2r)


# speedup vs baseline: 2.0905x; 2.0905x over previous
import functools

import jax
import jax.numpy as jnp
from jax import lax
from jax.experimental import pallas as pl
from jax.experimental.pallas import tpu as pltpu
from jax.experimental.pallas import tpu_sc as plsc

N = 10000
E = 160000
D = 256
H = 128
NT = 16
NP = 10240
STRIDE = NP // NT
EPT = E // NT
CH = 64
NCH = 159
ECT = NCH * CH
EP = NT * ECT
NBUF = 3

_mesh = plsc.VectorSubcoreMesh(core_axis_name="c", subcore_axis_name="s")

_sc_params = pltpu.CompilerParams(needs_layout_passes=False)


def _rsqrt16(x):
    x = jnp.maximum(x, 1.0)
    i = plsc.bitcast(x, jnp.int32)
    i = 0x5F3759DF - lax.shift_right_logical(i, 1)
    y = plsc.bitcast(i, jnp.float32)
    for _ in range(3):
        y = y * (1.5 - 0.5 * x * y * y)
    return y


@functools.partial(
    pl.kernel,
    out_type=[
        jax.ShapeDtypeStruct((NP,), jnp.float32),
        jax.ShapeDtypeStruct((EP,), jnp.float32),
    ],
    mesh=_mesh,
    scratch_types=[
        pltpu.VMEM((NP,), jnp.float32),
        pltpu.VMEM((ECT,), jnp.int32),
        pltpu.VMEM((ECT,), jnp.float32),
        pltpu.VMEM((STRIDE,), jnp.float32),
        pltpu.VMEM((STRIDE,), jnp.float32),
        pltpu.VMEM_SHARED((NT * NP,), jnp.float32),
    ],
    compiler_params=_sc_params,
)
def _prep_kernel(idx_hbm, srcf_hbm, wf_hbm, nd_hbm, c_hbm,
                 hist_v, idx_v, w_v, acc_v, tmp_v, shared):
    cx = lax.axis_index("c")
    s = lax.axis_index("s")
    pltpu.sync_copy(idx_hbm.at[pl.ds((cx * NT + s) * EPT, EPT)],
                    idx_v.at[pl.ds(0, EPT)])

    @pl.loop(0, NP, step=16)
    def _zero(i):
        hist_v[pl.ds(i, 16)] = jnp.zeros((16,), jnp.float32)

    @pl.loop(0, EPT, step=16)
    def _count(e0):
        idx16 = idx_v[pl.ds(e0, 16)]
        cnt, last = plsc.scan_count(idx16)
        plsc.addupdate_scatter(hist_v, [idx16], cnt.astype(jnp.float32),
                               mask=last)

    pltpu.sync_copy(hist_v, shared.at[pl.ds(s * NP, NP)])
    plsc.subcore_barrier()

    base = s * STRIDE

    @pl.loop(0, STRIDE, step=16)
    def _zacc(i):
        acc_v[pl.ds(i, 16)] = jnp.zeros((16,), jnp.float32)

    @pl.loop(0, NT)
    def _sum(t):
        pltpu.sync_copy(shared.at[pl.ds(t * NP + base, STRIDE)], tmp_v)

        @pl.loop(0, STRIDE, step=16)
        def _add(i):
            acc_v[pl.ds(i, 16)] = acc_v[pl.ds(i, 16)] + tmp_v[pl.ds(i, 16)]

    @pl.loop(0, STRIDE, step=16)
    def _norm(i):
        acc_v[pl.ds(i, 16)] = _rsqrt16(acc_v[pl.ds(i, 16)])

    plsc.subcore_barrier()

    @pl.when(cx == 1)
    def _():
        pltpu.sync_copy(acc_v, nd_hbm.at[pl.ds(base, STRIDE)])

    @pl.when(cx == 0)
    def _():
        pltpu.sync_copy(acc_v, shared.at[pl.ds(base, STRIDE)])

    plsc.subcore_barrier()

    @pl.when(cx == 0)
    def _():
        pltpu.sync_copy(shared.at[pl.ds(0, NP)], hist_v)
        pltpu.sync_copy(srcf_hbm.at[pl.ds(s * ECT, ECT)], idx_v)
        pltpu.sync_copy(wf_hbm.at[pl.ds(s * ECT, ECT)], w_v)

        @pl.loop(0, ECT, step=16)
        def _coef(e0):
            s16 = idx_v[pl.ds(e0, 16)]
            ns16 = plsc.load_gather(hist_v, [s16])
            w_v[pl.ds(e0, 16)] = w_v[pl.ds(e0, 16)] * ns16

        pltpu.sync_copy(w_v, c_hbm.at[pl.ds(s * ECT, ECT)])


def _make_spmm(fin):
    scratch = [
        pltpu.VMEM((NCH, CH), jnp.int32),
        pltpu.VMEM((NBUF, CH), jnp.int32),
        pltpu.VMEM((NBUF, CH), jnp.float32),
        [pltpu.VMEM((CH, H), jnp.float32)] * NBUF,
        pltpu.VMEM_SHARED((NP, H), jnp.float32),
        [pltpu.SemaphoreType.DMA] * NBUF,
        [pltpu.SemaphoreType.DMA] * NBUF,
    ]
    if fin:
        scratch += [pltpu.VMEM((STRIDE,), jnp.float32),
                    pltpu.VMEM((H,), jnp.float32)]
    out_type = jax.ShapeDtypeStruct((N, D) if fin else (2 * NP, H),
                                    jnp.float32)

    @functools.partial(pl.kernel, out_type=out_type, mesh=_mesh,
                       scratch_types=scratch, compiler_params=_sc_params)
    def _spmm(*args):
        if fin:
            (x_hbm, src_hbm, dst_hbm, w_hbm, zero_hbm, nd_hbm, b_hbm,
             out_hbm, src_v, dst_v, w_v, rows, acc, gsem, ssem,
             nd_v, b_v) = args
        else:
            (x_hbm, src_hbm, dst_hbm, w_hbm, zero_hbm, out_hbm,
             src_v, dst_v, w_v, rows, acc, gsem, ssem) = args
        _spmm_body(fin, x_hbm, src_hbm, dst_hbm, w_hbm, zero_hbm, out_hbm,
                   src_v, dst_v, w_v, rows, acc, gsem, ssem,
                   nd_hbm if fin else None, b_hbm if fin else None,
                   nd_v if fin else None, b_v if fin else None)

    return _spmm


def _spmm_body(fin, x_hbm, src_hbm, dst_hbm, w_hbm, zero_hbm, out_hbm,
               src_v, dst_v, w_v, rows, acc, gsem, ssem,
               nd_hbm, b_hbm, nd_v, b_v):
    c = lax.axis_index("c")
    s = lax.axis_index("s")
    pltpu.sync_copy(src_hbm.at[c * NT + s], src_v)

    def gstart(b, q):
        pltpu.async_copy(x_hbm.at[src_v.at[q]], rows[b], gsem[b])
        off = (s * NCH + q) * CH
        pltpu.async_copy(dst_hbm.at[pl.ds(off, CH)], dst_v.at[b], gsem[b])
        pltpu.async_copy(w_hbm.at[pl.ds(off, CH)], w_v.at[b], gsem[b])

    def gwait(b, q):
        pltpu.make_async_copy(x_hbm.at[src_v.at[q]], rows[b], gsem[b]).wait()
        off = (s * NCH + q) * CH
        pltpu.make_async_copy(dst_hbm.at[pl.ds(off, CH)], dst_v.at[b],
                              gsem[b]).wait()
        pltpu.make_async_copy(w_hbm.at[pl.ds(off, CH)], w_v.at[b],
                              gsem[b]).wait()

    def sstart(b):
        pltpu.async_copy(rows[b], acc.at[dst_v.at[b]], ssem[b], add=True)

    def swait(b):
        pltpu.make_async_copy(rows[b], acc.at[dst_v.at[b]], ssem[b]).wait()

    for b in range(NBUF - 1):
        gstart(b, b)

    pltpu.sync_copy(zero_hbm.at[pl.ds(s * STRIDE, STRIDE)],
                    acc.at[pl.ds(s * STRIDE, STRIDE)])
    plsc.subcore_barrier()

    @pl.loop(0, NCH, step=NBUF)
    def _iter(j):
        for k in range(NBUF):
            q = j + k
            b = k
            gwait(b, q)

            @pl.loop(0, CH, step=16)
            def _scale(r0):
                wv = w_v[b, pl.ds(r0, 16)]
                for r in range(16):
                    wr = wv[r]
                    for kk in range(H // 16):
                        sl = (r0 + r, pl.ds(kk * 16, 16))
                        rows[b][sl] = rows[b][sl] * wr

            sstart(b)
            nb = (k + NBUF - 1) % NBUF
            if k == 0:
                @pl.when(j > 0)
                def _():
                    swait(nb)

                gstart(nb, q + NBUF - 1)
            else:
                @pl.when(j < NCH - (NBUF - 1) - k)
                def _():
                    swait(nb)
                    gstart(nb, q + NBUF - 1)

    for b in range(NBUF):
        swait(b)
    plsc.subcore_barrier()

    if not fin:
        pltpu.sync_copy(acc.at[pl.ds(s * STRIDE, STRIDE)],
                        out_hbm.at[pl.ds(c * NP + s * STRIDE, STRIDE)])
        return

    pltpu.sync_copy(nd_hbm.at[pl.ds(s * STRIDE, STRIDE)], nd_v)
    pltpu.sync_copy(b_hbm.at[pl.ds(c * H, H)], b_v)
    bvs = [b_v[pl.ds(kk * 16, 16)] for kk in range(H // 16)]
    for blk in range(STRIDE // CH):
        row0 = s * STRIDE + blk * CH
        pltpu.sync_copy(acc.at[pl.ds(row0, CH)], rows[0])

        @pl.loop(0, CH, step=16)
        def _fscale(r0):
            ndv = nd_v[pl.ds(blk * CH + r0, 16)]
            for r in range(16):
                nr = ndv[r]
                for kk in range(H // 16):
                    sl = (r0 + r, pl.ds(kk * 16, 16))
                    rows[0][sl] = rows[0][sl] * nr + bvs[kk]

        if blk < 6:
            pltpu.sync_copy(rows[0],
                            out_hbm.at[pl.ds(row0, CH), pl.ds(c * H, H)])
        elif blk == 6:
            @pl.when(s < NT - 1)
            def _():
                pltpu.sync_copy(rows[0],
                                out_hbm.at[pl.ds(row0, CH), pl.ds(c * H, H)])

            @pl.when(s == NT - 1)
            def _():
                pltpu.sync_copy(
                    rows[0].at[pl.ds(0, N - (NT - 1) * STRIDE - 6 * CH)],
                    out_hbm.at[pl.ds((NT - 1) * STRIDE + 6 * CH,
                                     N - (NT - 1) * STRIDE - 6 * CH),
                               pl.ds(c * H, H)])
        else:
            @pl.when(s < NT - 1)
            def _():
                pltpu.sync_copy(rows[0],
                                out_hbm.at[pl.ds(row0, CH), pl.ds(c * H, H)])


_spmm1 = _make_spmm(False)
_spmm2 = _make_spmm(True)


BN = 1280


def _mm1_body(x_ref, w_ref, o_ref):
    x = x_ref[...]
    w = w_ref[...]
    o_ref[0] = jnp.dot(x, w[:, :H], preferred_element_type=jnp.float32)
    o_ref[1] = jnp.dot(x, w[:, H:], preferred_element_type=jnp.float32)


def _mm2_body(a_ref, nd_ref, b_ref, w_ref, o_ref):
    nd = nd_ref[...]
    b = b_ref[...]
    ha = jnp.maximum(a_ref[0] * nd + b[0, :H], 0.0)
    hb = jnp.maximum(a_ref[1] * nd + b[0, H:], 0.0)
    w = w_ref[...]
    o_ref[0] = (jnp.dot(ha, w[:H, :H], preferred_element_type=jnp.float32)
                + jnp.dot(hb, w[H:, :H], preferred_element_type=jnp.float32))
    o_ref[1] = (jnp.dot(ha, w[:H, H:], preferred_element_type=jnp.float32)
                + jnp.dot(hb, w[H:, H:], preferred_element_type=jnp.float32))


_row_spec = pl.BlockSpec((BN, D), lambda i: (i, 0))
_halves_spec = pl.BlockSpec((2, BN, H), lambda i: (0, i, 0))
_nd_spec = pl.BlockSpec((BN, 1), lambda i: (i, 0))
_w_spec = pl.BlockSpec((D, D), lambda i: (0, 0))
_b_spec = pl.BlockSpec((1, D), lambda i: (0, 0))

_mm1 = pl.pallas_call(
    _mm1_body,
    grid=(NP // BN,),
    in_specs=[_row_spec, _w_spec],
    out_specs=_halves_spec,
    out_shape=jax.ShapeDtypeStruct((2, NP, H), jnp.float32),
)

_mm2 = pl.pallas_call(
    _mm2_body,
    grid=(NP // BN,),
    in_specs=[_halves_spec, _nd_spec, _b_spec, _w_spec],
    out_specs=_halves_spec,
    out_shape=jax.ShapeDtypeStruct((2, NP, H), jnp.float32),
)

def kernel(features, edge_index, edge_weight, W1, b1, W2, b2):
    src = edge_index[0]
    dst = edge_index[1]

    idx_flat = jnp.concatenate([src, dst])
    pad = EP - E
    pad_idx = (jnp.arange(pad, dtype=jnp.int32) * 37) % N
    srcf = jnp.concatenate([src, pad_idx])
    srcp = srcf.reshape(NT, NCH, CH)
    srcp2 = jnp.concatenate([srcp, srcp + NP], axis=0)
    dstp = jnp.concatenate([dst, pad_idx])
    wp = jnp.concatenate([edge_weight, jnp.zeros((pad,), jnp.float32)])
    xpad = jnp.pad(features, ((0, NP - N), (0, 0)))
    zeros_half = jnp.zeros((NP, H), jnp.float32)
    b1r = b1.reshape(1, D)
    b2r = b2.reshape(1, D)

    nd, cw = _prep_kernel(idx_flat, srcf, wp)
    ndr = nd.reshape(NP, 1)
    x1 = _mm1(xpad, W1)
    a1 = _spmm1(x1.reshape(2 * NP, H), srcp2, dstp, cw, zeros_half)
    x2 = _mm2(a1.reshape(2, NP, H), ndr, b1r, W2)
    return _spmm2(x2.reshape(2 * NP, H), srcp2, dstp, cw, zeros_half,
                  nd, b2)


# speedup vs baseline: 2.1583x; 1.0324x over previous
import functools

import jax
import jax.numpy as jnp
from jax import lax
from jax.experimental import pallas as pl
from jax.experimental.pallas import tpu as pltpu
from jax.experimental.pallas import tpu_sc as plsc

N = 10000
E = 160000
D = 256
H = 128
NT = 16
NP = 10240
STRIDE = NP // NT
EPT = E // NT
CH = 64
NCH = 159
ECT = NCH * CH
EP = NT * ECT
NBUF = 3

_mesh = plsc.VectorSubcoreMesh(core_axis_name="c", subcore_axis_name="s")

_sc_params = pltpu.CompilerParams(needs_layout_passes=False)


def _rsqrt16(x):
    x = jnp.maximum(x, 1.0)
    i = plsc.bitcast(x, jnp.int32)
    i = 0x5F3759DF - lax.shift_right_logical(i, 1)
    y = plsc.bitcast(i, jnp.float32)
    for _ in range(3):
        y = y * (1.5 - 0.5 * x * y * y)
    return y


@functools.partial(
    pl.kernel,
    out_type=[
        jax.ShapeDtypeStruct((NP,), jnp.float32),
        jax.ShapeDtypeStruct((EP,), jnp.float32),
    ],
    mesh=_mesh,
    scratch_types=[
        pltpu.VMEM((NP,), jnp.float32),
        pltpu.VMEM((ECT,), jnp.int32),
        pltpu.VMEM((ECT,), jnp.float32),
        pltpu.VMEM((STRIDE,), jnp.float32),
        pltpu.VMEM((STRIDE,), jnp.float32),
        pltpu.VMEM_SHARED((NT * NP,), jnp.float32),
    ],
    compiler_params=_sc_params,
)
def _prep_kernel(idx_hbm, srcf_hbm, wf_hbm, nd_hbm, c_hbm,
                 hist_v, idx_v, w_v, acc_v, tmp_v, shared):
    cx = lax.axis_index("c")
    s = lax.axis_index("s")
    pltpu.sync_copy(idx_hbm.at[pl.ds((cx * NT + s) * EPT, EPT)],
                    idx_v.at[pl.ds(0, EPT)])

    @pl.loop(0, NP, step=16)
    def _zero(i):
        hist_v[pl.ds(i, 16)] = jnp.zeros((16,), jnp.float32)

    @pl.loop(0, EPT, step=16)
    def _count(e0):
        idx16 = idx_v[pl.ds(e0, 16)]
        cnt, last = plsc.scan_count(idx16)
        plsc.addupdate_scatter(hist_v, [idx16], cnt.astype(jnp.float32),
                               mask=last)

    pltpu.sync_copy(hist_v, shared.at[pl.ds(s * NP, NP)])
    plsc.subcore_barrier()

    base = s * STRIDE

    @pl.loop(0, STRIDE, step=16)
    def _zacc(i):
        acc_v[pl.ds(i, 16)] = jnp.zeros((16,), jnp.float32)

    @pl.loop(0, NT)
    def _sum(t):
        pltpu.sync_copy(shared.at[pl.ds(t * NP + base, STRIDE)], tmp_v)

        @pl.loop(0, STRIDE, step=16)
        def _add(i):
            acc_v[pl.ds(i, 16)] = acc_v[pl.ds(i, 16)] + tmp_v[pl.ds(i, 16)]

    @pl.loop(0, STRIDE, step=16)
    def _norm(i):
        acc_v[pl.ds(i, 16)] = _rsqrt16(acc_v[pl.ds(i, 16)])

    plsc.subcore_barrier()

    @pl.when(cx == 1)
    def _():
        pltpu.sync_copy(acc_v, nd_hbm.at[pl.ds(base, STRIDE)])

    @pl.when(cx == 0)
    def _():
        pltpu.sync_copy(acc_v, shared.at[pl.ds(base, STRIDE)])

    plsc.subcore_barrier()

    @pl.when(cx == 0)
    def _():
        pltpu.sync_copy(shared.at[pl.ds(0, NP)], hist_v)
        pltpu.sync_copy(srcf_hbm.at[pl.ds(s * ECT, ECT)], idx_v)
        pltpu.sync_copy(wf_hbm.at[pl.ds(s * ECT, ECT)], w_v)

        @pl.loop(0, ECT, step=16)
        def _coef(e0):
            s16 = idx_v[pl.ds(e0, 16)]
            ns16 = plsc.load_gather(hist_v, [s16])
            w_v[pl.ds(e0, 16)] = w_v[pl.ds(e0, 16)] * ns16

        pltpu.sync_copy(w_v, c_hbm.at[pl.ds(s * ECT, ECT)])


def _make_spmm(fin):
    scratch = [
        pltpu.VMEM((NCH, CH), jnp.int32),
        pltpu.VMEM((NBUF, CH), jnp.int32),
        pltpu.VMEM((NBUF, CH), jnp.float32),
        [pltpu.VMEM((CH, H), jnp.float32)] * NBUF,
        pltpu.VMEM_SHARED((NP, H), jnp.float32),
        [pltpu.SemaphoreType.DMA] * NBUF,
        [pltpu.SemaphoreType.DMA] * NBUF,
    ]
    if fin:
        scratch += [pltpu.VMEM((STRIDE,), jnp.float32),
                    pltpu.VMEM((H,), jnp.float32)]
    out_type = jax.ShapeDtypeStruct((N, D) if fin else (2 * NP, H),
                                    jnp.float32)

    @functools.partial(pl.kernel, out_type=out_type, mesh=_mesh,
                       scratch_types=scratch, compiler_params=_sc_params)
    def _spmm(*args):
        if fin:
            (x_hbm, src_hbm, dst_hbm, w_hbm, zero_hbm, nd_hbm, b_hbm,
             out_hbm, src_v, dst_v, w_v, rows, acc, gsem, ssem,
             nd_v, b_v) = args
        else:
            (x_hbm, src_hbm, dst_hbm, w_hbm, zero_hbm, out_hbm,
             src_v, dst_v, w_v, rows, acc, gsem, ssem) = args
        _spmm_body(fin, x_hbm, src_hbm, dst_hbm, w_hbm, zero_hbm, out_hbm,
                   src_v, dst_v, w_v, rows, acc, gsem, ssem,
                   nd_hbm if fin else None, b_hbm if fin else None,
                   nd_v if fin else None, b_v if fin else None)

    return _spmm


def _spmm_body(fin, x_hbm, src_hbm, dst_hbm, w_hbm, zero_hbm, out_hbm,
               src_v, dst_v, w_v, rows, acc, gsem, ssem,
               nd_hbm, b_hbm, nd_v, b_v):
    c = lax.axis_index("c")
    s = lax.axis_index("s")
    pltpu.sync_copy(src_hbm.at[c * NT + s], src_v)

    def gstart(b, q):
        pltpu.async_copy(x_hbm.at[src_v.at[q]], rows[b], gsem[b])
        off = (s * NCH + q) * CH
        pltpu.async_copy(dst_hbm.at[pl.ds(off, CH)], dst_v.at[b], gsem[b])
        pltpu.async_copy(w_hbm.at[pl.ds(off, CH)], w_v.at[b], gsem[b])

    def gwait(b, q):
        pltpu.make_async_copy(x_hbm.at[src_v.at[q]], rows[b], gsem[b]).wait()
        off = (s * NCH + q) * CH
        pltpu.make_async_copy(dst_hbm.at[pl.ds(off, CH)], dst_v.at[b],
                              gsem[b]).wait()
        pltpu.make_async_copy(w_hbm.at[pl.ds(off, CH)], w_v.at[b],
                              gsem[b]).wait()

    def sstart(b):
        pltpu.async_copy(rows[b], acc.at[dst_v.at[b]], ssem[b], add=True)

    def swait(b):
        pltpu.make_async_copy(rows[b], acc.at[dst_v.at[b]], ssem[b]).wait()

    for b in range(NBUF - 1):
        gstart(b, b)

    pltpu.sync_copy(zero_hbm.at[pl.ds(s * STRIDE, STRIDE)],
                    acc.at[pl.ds(s * STRIDE, STRIDE)])
    plsc.subcore_barrier()

    @pl.loop(0, NCH, step=NBUF)
    def _iter(j):
        for k in range(NBUF):
            q = j + k
            b = k
            gwait(b, q)
            nb = (k + NBUF - 1) % NBUF
            if k == 0:
                @pl.when(j > 0)
                def _():
                    swait(nb)

                gstart(nb, q + NBUF - 1)
            else:
                @pl.when(j < NCH - (NBUF - 1) - k)
                def _():
                    swait(nb)
                    gstart(nb, q + NBUF - 1)

            @pl.loop(0, CH, step=16)
            def _scale(r0):
                wv = w_v[b, pl.ds(r0, 16)]
                for r in range(16):
                    wr = wv[r]
                    for kk in range(H // 16):
                        sl = (r0 + r, pl.ds(kk * 16, 16))
                        rows[b][sl] = rows[b][sl] * wr

            sstart(b)

    for b in range(NBUF):
        swait(b)
    plsc.subcore_barrier()

    if not fin:
        pltpu.sync_copy(acc.at[pl.ds(s * STRIDE, STRIDE)],
                        out_hbm.at[pl.ds(c * NP + s * STRIDE, STRIDE)])
        return

    pltpu.sync_copy(nd_hbm.at[pl.ds(s * STRIDE, STRIDE)], nd_v)
    pltpu.sync_copy(b_hbm.at[pl.ds(c * H, H)], b_v)
    bvs = [b_v[pl.ds(kk * 16, 16)] for kk in range(H // 16)]
    for blk in range(STRIDE // CH):
        row0 = s * STRIDE + blk * CH
        pltpu.sync_copy(acc.at[pl.ds(row0, CH)], rows[0])

        @pl.loop(0, CH, step=16)
        def _fscale(r0):
            ndv = nd_v[pl.ds(blk * CH + r0, 16)]
            for r in range(16):
                nr = ndv[r]
                for kk in range(H // 16):
                    sl = (r0 + r, pl.ds(kk * 16, 16))
                    rows[0][sl] = rows[0][sl] * nr + bvs[kk]

        if blk < 6:
            pltpu.sync_copy(rows[0],
                            out_hbm.at[pl.ds(row0, CH), pl.ds(c * H, H)])
        elif blk == 6:
            @pl.when(s < NT - 1)
            def _():
                pltpu.sync_copy(rows[0],
                                out_hbm.at[pl.ds(row0, CH), pl.ds(c * H, H)])

            @pl.when(s == NT - 1)
            def _():
                pltpu.sync_copy(
                    rows[0].at[pl.ds(0, N - (NT - 1) * STRIDE - 6 * CH)],
                    out_hbm.at[pl.ds((NT - 1) * STRIDE + 6 * CH,
                                     N - (NT - 1) * STRIDE - 6 * CH),
                               pl.ds(c * H, H)])
        else:
            @pl.when(s < NT - 1)
            def _():
                pltpu.sync_copy(rows[0],
                                out_hbm.at[pl.ds(row0, CH), pl.ds(c * H, H)])


_spmm1 = _make_spmm(False)
_spmm2 = _make_spmm(True)


BN = 1280


def _mm1_body(x_ref, w_ref, o_ref):
    x = x_ref[...]
    w = w_ref[...]
    o_ref[0] = jnp.dot(x, w[:, :H], preferred_element_type=jnp.float32)
    o_ref[1] = jnp.dot(x, w[:, H:], preferred_element_type=jnp.float32)


def _mm2_body(a_ref, nd_ref, b_ref, w_ref, o_ref):
    nd = nd_ref[...]
    b = b_ref[...]
    ha = jnp.maximum(a_ref[0] * nd + b[0, :H], 0.0)
    hb = jnp.maximum(a_ref[1] * nd + b[0, H:], 0.0)
    w = w_ref[...]
    o_ref[0] = (jnp.dot(ha, w[:H, :H], preferred_element_type=jnp.float32)
                + jnp.dot(hb, w[H:, :H], preferred_element_type=jnp.float32))
    o_ref[1] = (jnp.dot(ha, w[:H, H:], preferred_element_type=jnp.float32)
                + jnp.dot(hb, w[H:, H:], preferred_element_type=jnp.float32))


_row_spec = pl.BlockSpec((BN, D), lambda i: (i, 0))
_halves_spec = pl.BlockSpec((2, BN, H), lambda i: (0, i, 0))
_nd_spec = pl.BlockSpec((BN, 1), lambda i: (i, 0))
_w_spec = pl.BlockSpec((D, D), lambda i: (0, 0))
_b_spec = pl.BlockSpec((1, D), lambda i: (0, 0))

_mm1 = pl.pallas_call(
    _mm1_body,
    grid=(NP // BN,),
    in_specs=[_row_spec, _w_spec],
    out_specs=_halves_spec,
    out_shape=jax.ShapeDtypeStruct((2, NP, H), jnp.float32),
)

_mm2 = pl.pallas_call(
    _mm2_body,
    grid=(NP // BN,),
    in_specs=[_halves_spec, _nd_spec, _b_spec, _w_spec],
    out_specs=_halves_spec,
    out_shape=jax.ShapeDtypeStruct((2, NP, H), jnp.float32),
)

def kernel(features, edge_index, edge_weight, W1, b1, W2, b2):
    src = edge_index[0]
    dst = edge_index[1]

    idx_flat = jnp.concatenate([src, dst])
    pad = EP - E
    pad_idx = (jnp.arange(pad, dtype=jnp.int32) * 37) % N
    srcf = jnp.concatenate([src, pad_idx])
    srcp = srcf.reshape(NT, NCH, CH)
    srcp2 = jnp.concatenate([srcp, srcp + NP], axis=0)
    dstp = jnp.concatenate([dst, pad_idx])
    wp = jnp.concatenate([edge_weight, jnp.zeros((pad,), jnp.float32)])
    xpad = jnp.pad(features, ((0, NP - N), (0, 0)))
    zeros_half = jnp.zeros((NP, H), jnp.float32)
    b1r = b1.reshape(1, D)
    b2r = b2.reshape(1, D)

    nd, cw = _prep_kernel(idx_flat, srcf, wp)
    ndr = nd.reshape(NP, 1)
    x1 = _mm1(xpad, W1)
    a1 = _spmm1(x1.reshape(2 * NP, H), srcp2, dstp, cw, zeros_half)
    x2 = _mm2(a1.reshape(2, NP, H), ndr, b1r, W2)
    return _spmm2(x2.reshape(2 * NP, H), srcp2, dstp, cw, zeros_half,
                  nd, b2)
